# baseline (device time: 227695 ns/iter reference)
import jax
import jax.numpy as jnp
from jax import lax
from jax.experimental import pallas as pl
from jax.experimental.pallas import tpu as pltpu

N_DEV = 16


def kernel(x, w_mat, scale_x, scale_w):
    m_per, k = x.shape
    _, n_per = w_mat.shape

    def body(x_ref, w_ref, sx_ref, sw_ref, out_ref, comm_ref,
             send_sems, recv_sems):
        my = lax.axis_index("i")
        left = lax.rem(my - 1 + N_DEV, N_DEV)
        right = lax.rem(my + 1, N_DEV)

        barrier_sem = pltpu.get_barrier_semaphore()
        for nbr in (left, right):
            pl.semaphore_signal(
                barrier_sem, inc=1,
                device_id=(nbr,), device_id_type=pl.DeviceIdType.MESH,
            )
        pl.semaphore_wait(barrier_sem, 2)

        scale = sx_ref[0] * sw_ref[0]

        def gemm_store(chunk, origin):
            acc = lax.dot_general(
                chunk, w_ref[...],
                (((1,), (0,)), ((), ())),
                preferred_element_type=jnp.int32,
            )
            out_ref[pl.ds(origin * m_per, m_per), :] = (
                acc.astype(jnp.float32) * scale
            )

        comm_ref[0] = x_ref[...]
        gemm_store(x_ref[...], my)

        for h in range(N_DEV - 1):
            rdma = pltpu.make_async_remote_copy(
                src_ref=comm_ref.at[h],
                dst_ref=comm_ref.at[h + 1],
                send_sem=send_sems.at[h],
                recv_sem=recv_sems.at[h],
                device_id=(right,),
                device_id_type=pl.DeviceIdType.MESH,
            )
            rdma.start()
            rdma.wait()
            origin = lax.rem(my - h - 1 + N_DEV, N_DEV)
            gemm_store(comm_ref[h + 1], origin)

    return pl.pallas_call(
        body,
        out_shape=jax.ShapeDtypeStruct((N_DEV * m_per, n_per), jnp.float32),
        in_specs=[
            pl.BlockSpec(memory_space=pltpu.VMEM),
            pl.BlockSpec(memory_space=pltpu.VMEM),
            pl.BlockSpec(memory_space=pltpu.SMEM),
            pl.BlockSpec(memory_space=pltpu.SMEM),
        ],
        out_specs=pl.BlockSpec(memory_space=pltpu.VMEM),
        scratch_shapes=[
            pltpu.VMEM((N_DEV, m_per, k), jnp.int8),
            pltpu.SemaphoreType.DMA((N_DEV - 1,)),
            pltpu.SemaphoreType.DMA((N_DEV - 1,)),
        ],
        compiler_params=pltpu.CompilerParams(collective_id=0),
    )(x, w_mat, scale_x, scale_w)


# device time: 116170 ns/iter; 1.9600x vs baseline; 1.9600x over previous
import jax
import jax.numpy as jnp
from jax import lax
from jax.experimental import pallas as pl
from jax.experimental.pallas import tpu as pltpu

N_DEV = 16
N_FWD = 8
N_BWD = 7

RING = (0, 1, 5, 9, 13, 14, 10, 6, 2, 3, 7, 11, 15, 12, 8, 4)
POS = tuple(RING.index(l) for l in range(N_DEV))


def kernel(x, w_mat, scale_x, scale_w):
    m_per, k = x.shape
    _, n_per = w_mat.shape

    def body(x_ref, w_ref, sx_ref, sw_ref, ring_ref, pos_ref, out_ref,
             fwd_comm, bwd_comm, fwd_send, fwd_recv, bwd_send, bwd_recv):
        my = lax.axis_index("i")

        p = pos_ref[my]
        nxt = ring_ref[lax.rem(p + 1, N_DEV)]
        prv = ring_ref[lax.rem(p - 1 + N_DEV, N_DEV)]

        def origin_fwd(s):
            return ring_ref[lax.rem(p - s + 2 * N_DEV, N_DEV)]

        def origin_bwd(s):
            return ring_ref[lax.rem(p + s, N_DEV)]

        barrier_sem = pltpu.get_barrier_semaphore()
        for nbr in (nxt, prv):
            pl.semaphore_signal(
                barrier_sem, inc=1,
                device_id=(nbr,), device_id_type=pl.DeviceIdType.MESH,
            )
        pl.semaphore_wait(barrier_sem, 2)

        scale = sx_ref[0] * sw_ref[0]

        def gemm_store(chunk, origin):
            acc = lax.dot_general(
                chunk, w_ref[...],
                (((1,), (0,)), ((), ())),
                preferred_element_type=jnp.int32,
            )
            out_ref[pl.ds(origin * m_per, m_per), :] = (
                acc.astype(jnp.float32) * scale
            )

        def make_fwd(h):
            return pltpu.make_async_remote_copy(
                src_ref=x_ref if h == 0 else fwd_comm.at[h],
                dst_ref=fwd_comm.at[h + 1],
                send_sem=fwd_send.at[h],
                recv_sem=fwd_recv.at[h],
                device_id=(nxt,),
                device_id_type=pl.DeviceIdType.MESH,
            )

        def make_bwd(h):
            return pltpu.make_async_remote_copy(
                src_ref=x_ref if h == 0 else bwd_comm.at[h],
                dst_ref=bwd_comm.at[h + 1],
                send_sem=bwd_send.at[h],
                recv_sem=bwd_recv.at[h],
                device_id=(prv,),
                device_id_type=pl.DeviceIdType.MESH,
            )

        fwd = [make_fwd(h) for h in range(N_FWD)]
        bwd = [make_bwd(h) for h in range(N_BWD)]

        fwd[0].start()
        bwd[0].start()
        gemm_store(x_ref[...], my)

        for h in range(1, N_FWD + 1):
            fwd[h - 1].wait_recv()
            if h < N_FWD:
                fwd[h].start()
            if h <= N_BWD:
                bwd[h - 1].wait_recv()
                if h < N_BWD:
                    bwd[h].start()
            gemm_store(fwd_comm[h], origin_fwd(h))
            if h <= N_BWD:
                gemm_store(bwd_comm[h], origin_bwd(h))

        for r in fwd + bwd:
            r.wait_send()

    return pl.pallas_call(
        body,
        out_shape=jax.ShapeDtypeStruct((N_DEV * m_per, n_per), jnp.float32),
        in_specs=[
            pl.BlockSpec(memory_space=pltpu.VMEM),
            pl.BlockSpec(memory_space=pltpu.VMEM),
            pl.BlockSpec(memory_space=pltpu.SMEM),
            pl.BlockSpec(memory_space=pltpu.SMEM),
            pl.BlockSpec(memory_space=pltpu.SMEM),
            pl.BlockSpec(memory_space=pltpu.SMEM),
        ],
        out_specs=pl.BlockSpec(memory_space=pltpu.VMEM),
        scratch_shapes=[
            pltpu.VMEM((N_FWD + 1, m_per, k), jnp.int8),
            pltpu.VMEM((N_BWD + 1, m_per, k), jnp.int8),
            pltpu.SemaphoreType.DMA((N_FWD,)),
            pltpu.SemaphoreType.DMA((N_FWD,)),
            pltpu.SemaphoreType.DMA((N_BWD,)),
            pltpu.SemaphoreType.DMA((N_BWD,)),
        ],
        compiler_params=pltpu.CompilerParams(collective_id=0),
    )(x, w_mat, scale_x, scale_w,
      jnp.array(RING, jnp.int32), jnp.array(POS, jnp.int32))


# device time: 98822 ns/iter; 2.3041x vs baseline; 1.1755x over previous
import jax
import jax.numpy as jnp
from jax import lax
from jax.experimental import pallas as pl
from jax.experimental.pallas import tpu as pltpu

N_DEV = 16
N_HOP = 8

RING = (0, 1, 5, 9, 13, 14, 10, 6, 2, 3, 7, 11, 15, 12, 8, 4)
POS = tuple(RING.index(l) for l in range(N_DEV))


def kernel(x, w_mat, scale_x, scale_w):
    m_per, k = x.shape
    _, n_per = w_mat.shape
    m_half = m_per // 2

    def body(x_ref, w_ref, sx_ref, sw_ref, ring_ref, pos_ref, out_ref,
             fwd_comm, bwd_comm, fwd_send, fwd_recv, bwd_send, bwd_recv):
        my = lax.axis_index("i")

        p = pos_ref[my]
        nxt = ring_ref[lax.rem(p + 1, N_DEV)]
        prv = ring_ref[lax.rem(p - 1 + N_DEV, N_DEV)]

        def origin_fwd(s):
            return ring_ref[lax.rem(p - s + 2 * N_DEV, N_DEV)]

        def origin_bwd(s):
            return ring_ref[lax.rem(p + s, N_DEV)]

        barrier_sem = pltpu.get_barrier_semaphore()
        for nbr in (nxt, prv):
            pl.semaphore_signal(
                barrier_sem, inc=1,
                device_id=(nbr,), device_id_type=pl.DeviceIdType.MESH,
            )
        pl.semaphore_wait(barrier_sem, 2)

        scale = sx_ref[0] * sw_ref[0]

        def gemm_store(chunk, origin, row_off, rows):
            acc = lax.dot_general(
                chunk, w_ref[...],
                (((1,), (0,)), ((), ())),
                preferred_element_type=jnp.int32,
            )
            out_ref[pl.ds(origin * m_per + row_off, rows), :] = (
                acc.astype(jnp.float32) * scale
            )

        def half_slice(ref, half):
            return ref.at[pl.ds(half * m_half, m_half), :]

        def make(h, half, comm, send_sems, recv_sems, dev):
            src = x_ref if h == 0 else comm.at[h]
            return pltpu.make_async_remote_copy(
                src_ref=half_slice(src, half),
                dst_ref=half_slice(comm.at[h + 1], half),
                send_sem=send_sems.at[h, half],
                recv_sem=recv_sems.at[h, half],
                device_id=(dev,),
                device_id_type=pl.DeviceIdType.MESH,
            )

        fwd = {(h, hf): make(h, hf, fwd_comm, fwd_send, fwd_recv, nxt)
               for h in range(N_HOP) for hf in (0, 1)
               if not (h == N_HOP - 1 and hf == 1)}
        bwd = {(h, hf): make(h, hf, bwd_comm, bwd_send, bwd_recv, prv)
               for h in range(N_HOP) for hf in (0, 1)
               if not (h == N_HOP - 1 and hf == 0)}

        fwd[(0, 0)].start()
        bwd[(0, 1)].start()
        fwd[(0, 1)].start()
        bwd[(0, 0)].start()
        gemm_store(x_ref[...], my, 0, m_per)

        for h in range(1, N_HOP + 1):
            fwd[(h - 1, 0)].wait_recv()
            if h < N_HOP:
                fwd[(h, 0)].start()
            bwd[(h - 1, 1)].wait_recv()
            if h < N_HOP:
                bwd[(h, 1)].start()
            if h < N_HOP:
                fwd[(h - 1, 1)].wait_recv()
                if h < N_HOP - 1:
                    fwd[(h, 1)].start()
                bwd[(h - 1, 0)].wait_recv()
                if h < N_HOP - 1:
                    bwd[(h, 0)].start()
            if h < N_HOP:
                gemm_store(fwd_comm[h], origin_fwd(h), 0, m_per)
                gemm_store(bwd_comm[h], origin_bwd(h), 0, m_per)
            else:
                gemm_store(fwd_comm[h, : m_half], origin_fwd(h), 0, m_half)
                gemm_store(bwd_comm[h, m_half :], origin_bwd(h), m_half,
                           m_half)

        for r in list(fwd.values()) + list(bwd.values()):
            r.wait_send()

    return pl.pallas_call(
        body,
        out_shape=jax.ShapeDtypeStruct((N_DEV * m_per, n_per), jnp.float32),
        in_specs=[
            pl.BlockSpec(memory_space=pltpu.VMEM),
            pl.BlockSpec(memory_space=pltpu.VMEM),
            pl.BlockSpec(memory_space=pltpu.SMEM),
            pl.BlockSpec(memory_space=pltpu.SMEM),
            pl.BlockSpec(memory_space=pltpu.SMEM),
            pl.BlockSpec(memory_space=pltpu.SMEM),
        ],
        out_specs=pl.BlockSpec(memory_space=pltpu.VMEM),
        scratch_shapes=[
            pltpu.VMEM((N_HOP + 1, m_per, k), jnp.int8),
            pltpu.VMEM((N_HOP + 1, m_per, k), jnp.int8),
            pltpu.SemaphoreType.DMA((N_HOP, 2)),
            pltpu.SemaphoreType.DMA((N_HOP, 2)),
            pltpu.SemaphoreType.DMA((N_HOP, 2)),
            pltpu.SemaphoreType.DMA((N_HOP, 2)),
        ],
        compiler_params=pltpu.CompilerParams(collective_id=0),
    )(x, w_mat, scale_x, scale_w,
      jnp.array(RING, jnp.int32), jnp.array(POS, jnp.int32))


# device time: 87196 ns/iter; 2.6113x vs baseline; 1.1333x over previous
import jax
import jax.numpy as jnp
from jax import lax
from jax.experimental import pallas as pl
from jax.experimental.pallas import tpu as pltpu

N_DEV = 16
N_Z = 4
N_W = 4


def kernel(x, w_mat, scale_x, scale_w):
    m_per, k = x.shape
    _, n_per = w_mat.shape
    m_half = m_per // 2

    def body(x_ref, w_ref, sx_ref, sw_ref, out_ref,
             col_buf, full_cw, full_ccw, half_a, half_b,
             zs_up, zr_up, zs_dn, zr_dn,
             s_cwf, r_cwf, s_ccwf, r_ccwf,
             s_cwh, r_cwh, s_ccwh, r_ccwh):
        my = lax.axis_index("i")
        z = lax.div(my, N_W)
        w = lax.rem(my, N_W)
        cw_dev = N_W * z + lax.rem(w + 1, N_W)
        ccw_dev = N_W * z + lax.rem(w + 3, N_W)
        up_dev = N_W * jnp.minimum(z + 1, N_Z - 1) + w
        dn_dev = N_W * jnp.maximum(z - 1, 0) + w
        has_up = z < N_Z - 1
        has_dn = z > 0

        barrier_sem = pltpu.get_barrier_semaphore()
        for tgt in (cw_dev, ccw_dev):
            pl.semaphore_signal(
                barrier_sem, inc=1,
                device_id=(tgt,), device_id_type=pl.DeviceIdType.MESH,
            )

        @pl.when(has_up)
        def _():
            pl.semaphore_signal(
                barrier_sem, inc=1,
                device_id=(up_dev,), device_id_type=pl.DeviceIdType.MESH,
            )

        @pl.when(has_dn)
        def _():
            pl.semaphore_signal(
                barrier_sem, inc=1,
                device_id=(dn_dev,), device_id_type=pl.DeviceIdType.MESH,
            )

        both_z = jnp.logical_and(has_up, has_dn)

        @pl.when(both_z)
        def _():
            pl.semaphore_wait(barrier_sem, 4)

        @pl.when(jnp.logical_not(both_z))
        def _():
            pl.semaphore_wait(barrier_sem, 3)

        scale = sx_ref[0] * sw_ref[0]

        def gemm_store(chunk, origin, row_off, rows):
            acc = lax.dot_general(
                chunk, w_ref[...],
                (((1,), (0,)), ((), ())),
                preferred_element_type=jnp.int32,
            )
            out_ref[pl.ds(origin * m_per + row_off, rows), :] = (
                acc.astype(jnp.float32) * scale
            )

        def rc(src, dst, ssem, rsem, dev):
            return pltpu.make_async_remote_copy(
                src_ref=src, dst_ref=dst, send_sem=ssem, recv_sem=rsem,
                device_id=(dev,), device_id_type=pl.DeviceIdType.MESH,
            )

        zup = [rc(col_buf.at[s], col_buf.at[s],
                  zs_up.at[s], zr_up.at[s], up_dev) for s in range(N_Z)]
        zupo = [rc(x_ref, col_buf.at[s],
                   zs_up.at[s], zr_up.at[s], up_dev) for s in range(N_Z)]
        zdn = [rc(col_buf.at[s], col_buf.at[s],
                  zs_dn.at[s], zr_dn.at[s], dn_dev) for s in range(N_Z)]
        zdno = [rc(x_ref, col_buf.at[s],
                   zs_dn.at[s], zr_dn.at[s], dn_dev) for s in range(N_Z)]
        cwf = [rc(col_buf.at[s], full_cw.at[s],
                  s_cwf.at[s], r_cwf.at[s], cw_dev) for s in range(N_Z)]
        cwfo = [rc(x_ref, full_cw.at[s],
                   s_cwf.at[s], r_cwf.at[s], cw_dev) for s in range(N_Z)]
        ccwf = [rc(col_buf.at[s], full_ccw.at[s],
                   s_ccwf.at[s], r_ccwf.at[s], ccw_dev) for s in range(N_Z)]
        ccwfo = [rc(x_ref, full_ccw.at[s],
                    s_ccwf.at[s], r_ccwf.at[s], ccw_dev) for s in range(N_Z)]
        cwh = [rc(full_cw.at[s, pl.ds(0, m_half), :], half_a.at[s],
                  s_cwh.at[s], r_cwh.at[s], cw_dev) for s in range(N_Z)]
        ccwh = [rc(full_ccw.at[s, pl.ds(m_half, m_half), :], half_b.at[s],
                   s_ccwh.at[s], r_ccwh.at[s], ccw_dev) for s in range(N_Z)]

        for s in range(N_Z):
            mine = z == s

            @pl.when(mine)
            def _(s=s):
                cwfo[s].start()
                ccwfo[s].start()

            @pl.when(jnp.logical_and(mine, has_up))
            def _(s=s):
                zupo[s].start()

            @pl.when(jnp.logical_and(mine, has_dn))
            def _(s=s):
                zdno[s].start()

        gemm_store(x_ref[...], my, 0, m_per)

        for r in range(1, N_Z):
            for s in range(N_Z - 1):
                pred = z == s + r

                @pl.when(pred)
                def _(s=s):
                    zup[s].wait_recv()

                @pl.when(jnp.logical_and(pred, has_up))
                def _(s=s):
                    zup[s].start()

                @pl.when(pred)
                def _(s=s):
                    cwf[s].start()
                    ccwf[s].start()
                    gemm_store(col_buf[s], N_W * s + w, 0, m_per)

            for s in range(1, N_Z):
                pred = z == s - r

                @pl.when(pred)
                def _(s=s):
                    zdn[s].wait_recv()

                @pl.when(jnp.logical_and(pred, has_dn))
                def _(s=s):
                    zdn[s].start()

                @pl.when(pred)
                def _(s=s):
                    cwf[s].start()
                    ccwf[s].start()
                    gemm_store(col_buf[s], N_W * s + w, 0, m_per)

        for s in range(N_Z):
            cwf[s].wait_recv()
            cwh[s].start()
            gemm_store(full_cw[s], N_W * s + lax.rem(w + 3, N_W), 0, m_per)
            ccwf[s].wait_recv()
            ccwh[s].start()
            gemm_store(full_ccw[s], N_W * s + lax.rem(w + 1, N_W), 0, m_per)

        for s in range(N_Z):
            diag = N_W * s + lax.rem(w + 2, N_W)
            cwh[s].wait_recv()
            gemm_store(half_a[s], diag, 0, m_half)
            ccwh[s].wait_recv()
            gemm_store(half_b[s], diag, m_half, m_half)

        for s in range(N_Z):
            @pl.when(jnp.logical_and(s <= z, has_up))
            def _(s=s):
                zup[s].wait_send()

            @pl.when(jnp.logical_and(s >= z, has_dn))
            def _(s=s):
                zdn[s].wait_send()

            cwf[s].wait_send()
            ccwf[s].wait_send()
            cwh[s].wait_send()
            ccwh[s].wait_send()

    return pl.pallas_call(
        body,
        out_shape=jax.ShapeDtypeStruct((N_DEV * m_per, n_per), jnp.float32),
        in_specs=[
            pl.BlockSpec(memory_space=pltpu.VMEM),
            pl.BlockSpec(memory_space=pltpu.VMEM),
            pl.BlockSpec(memory_space=pltpu.SMEM),
            pl.BlockSpec(memory_space=pltpu.SMEM),
        ],
        out_specs=pl.BlockSpec(memory_space=pltpu.VMEM),
        scratch_shapes=[
            pltpu.VMEM((N_Z, m_per, k), jnp.int8),
            pltpu.VMEM((N_Z, m_per, k), jnp.int8),
            pltpu.VMEM((N_Z, m_per, k), jnp.int8),
            pltpu.VMEM((N_Z, m_half, k), jnp.int8),
            pltpu.VMEM((N_Z, m_half, k), jnp.int8),
        ] + [pltpu.SemaphoreType.DMA((N_Z,))] * 12,
        compiler_params=pltpu.CompilerParams(collective_id=0),
    )(x, w_mat, scale_x, scale_w)
